# Initial kernel scaffold; baseline (speedup 1.0000x reference)
import jax
import jax.numpy as jnp
from jax.experimental import pallas as pl


def kernel(xyz, points, former_fps_idx, vote_ctr, W0_0, b0_0, W0_1, b0_1, W0_2, b0_2, W1_0, b1_0, W1_1, b1_1, W1_2, b1_2, W2_0, b2_0, W2_1, b2_1, W2_2, b2_2, W_agg, b_agg):
    raise NotImplementedError("write your pallas kernel here")


# speedup vs baseline: 26.1938x; 26.1938x over previous
import jax
import jax.numpy as jnp
from jax import lax
from jax.experimental import pallas as pl
from jax.experimental.pallas import tpu as pltpu
from jax.experimental.pallas import tpu_sc as plsc

B = 4
N = 4096
NPOINT = 512
S = 576
ROWS = B * S
DP = 48
CIN = 35
NSS = (16, 32, 64)
RSQ = (0.2 * 0.2, 0.4 * 0.4, 0.8 * 0.8)
NW = 32
RPW = ROWS // NW
G = 12
NG = RPW // G
RB = 128
CHUNKS = N // 16


def _sc_body(xh, yh, zh, vidxh, voteh, feath,
             nxzh, g0h, g1h, g2h,
             xv, yv, zv, x2v, vteb, cand0, cand1, cand2,
             land0, land1, land2, vidxv, cxr, cyr, czr, sem):
    cid = lax.axis_index("c")
    sid = lax.axis_index("s")
    wid = sid * 2 + cid
    b = wid // (S // RPW)
    base = wid * RPW
    pltpu.sync_copy(xh.at[b], xv)
    pltpu.sync_copy(yh.at[b], yv)
    pltpu.sync_copy(zh.at[b], zv)
    pltpu.sync_copy(vidxh.at[pl.ds(base, RPW)], vidxv.at[pl.ds(0, RPW)])
    pltpu.sync_copy(voteh.at[pl.ds(b * NPOINT * 8, NPOINT * 8)], vteb)
    boff = b * N
    iota = lax.iota(jnp.int32, 16)
    zero16 = jnp.zeros((16,), jnp.int32)
    boffv = jnp.broadcast_to(boff, (16,))
    cands = (cand0, cand1, cand2)

    def bf16r(v):
        u = plsc.bitcast(v, jnp.int32)
        u = (u + 0x7FFF + ((u >> 16) & 1)) & ~0xFFFF
        return plsc.bitcast(u, jnp.float32)

    def pre_body(c, _):
        sl = pl.ds(c * 16, 16)
        px, py, pz = xv[sl], yv[sl], zv[sl]
        x2v[sl] = (px * px + py * py) + pz * pz
        xv[sl] = bf16r(px)
        yv[sl] = bf16r(py)
        zv[sl] = bf16r(pz)
        return 0

    lax.fori_loop(0, CHUNKS, pre_body, 0)

    for q in range(RPW // 16 + 1):
        sl = pl.ds(q * 16, 16)
        vv = vidxv[sl]
        vv = jnp.minimum(jnp.maximum(vv, 0), NPOINT - 1) * 8
        cxr[sl] = plsc.load_gather(vteb, [vv])
        cyr[sl] = plsc.load_gather(vteb, [vv + 1])
        czr[sl] = plsc.load_gather(vteb, [vv + 2])

    def row_body(r, _):
        ridx = jnp.broadcast_to(r, (16,))
        cx = plsc.load_gather(cxr, [ridx])
        cy = plsc.load_gather(cyr, [ridx])
        cz = plsc.load_gather(czr, [ridx])
        c2 = (cx * cx + cy * cy) + cz * cz
        cxb = bf16r(cx)
        cyb = bf16r(cy)
        czb = bf16r(cz)

        def chunk(c, carry):
            sl = pl.ds(c * 16, 16)
            dot = (cxb * xv[sl] + cyb * yv[sl]) + czb * zv[sl]
            d = (c2 + x2v[sl]) - 2.0 * dot
            gi = iota + (c * 16 + boff)
            new = []
            for k in range(3):
                ns = NSS[k]
                m = d <= jnp.float32(RSQ[k])
                cnt = carry[k]
                posr = cnt + plsc.cumsum(m.astype(jnp.int32)) - 1
                mst = jnp.logical_and(m, posr < ns)
                pos = jnp.minimum(posr, ns - 1) + r * ns
                plsc.store_scatter(cands[k], [pos], gi, mask=mst)
                new.append(jnp.minimum(
                    cnt + plsc.all_reduce_population_count(m), ns))
            return tuple(new)

        cnts = lax.fori_loop(0, CHUNKS, chunk, (zero16, zero16, zero16))
        for k in range(3):
            ns = NSS[k]
            cnt = cnts[k]
            first = plsc.load_gather(cands[k], [zero16 + r * ns])
            sel = jnp.where(cnt > 0, first, boffv)
            for q in range(ns // 16):
                slq = pl.ds(r * ns + q * 16, 16)
                keep = (iota + q * 16) < cnt
                cands[k][slq] = jnp.where(keep, cands[k][slq], sel)
        return 0

    lax.fori_loop(0, RPW, row_body, 0)

    ghs = (g0h, g1h, g2h)
    lands = (land0, land1, land2)

    def group(gidx, _):
        row0 = gidx * G
        cps = []
        for k in range(3):
            ns = NSS[k]
            tot = G * ns
            nch = (tot + 127) // 128
            csz = tot // nch
            for j in range(nch):
                idx_ref = cands[k].at[pl.ds(row0 * ns + j * csz, csz)]
                dst = lands[k].at[pl.ds(j * csz, csz)]
                cps.append(pltpu.async_copy(feath.at[idx_ref], dst, sem))
        for cp in cps:
            cp.wait()
        for k in range(3):
            ns = NSS[k]
            pltpu.sync_copy(lands[k],
                            ghs[k].at[pl.ds((base + row0) * ns, G * ns)])
        return 0

    lax.fori_loop(0, NG, group, 0)
    pltpu.sync_copy(cxr.at[pl.ds(0, RPW)], nxzh.at[pl.ds(base, RPW)])
    pltpu.sync_copy(cyr.at[pl.ds(0, RPW)], nxzh.at[pl.ds(ROWS + base, RPW)])
    pltpu.sync_copy(czr.at[pl.ds(0, RPW)],
                    nxzh.at[pl.ds(2 * ROWS + base, RPW)])


_SC_OUT = (
    jax.ShapeDtypeStruct((3 * ROWS,), jnp.float32),
    jax.ShapeDtypeStruct((ROWS * 16, DP), jnp.float32),
    jax.ShapeDtypeStruct((ROWS * 32, DP), jnp.float32),
    jax.ShapeDtypeStruct((ROWS * 64, DP), jnp.float32),
)

_SC_SCRATCH = [
    pltpu.VMEM((N,), jnp.float32),
    pltpu.VMEM((N,), jnp.float32),
    pltpu.VMEM((N,), jnp.float32),
    pltpu.VMEM((N,), jnp.float32),
    pltpu.VMEM((NPOINT * 8,), jnp.float32),
    pltpu.VMEM((RPW * 16,), jnp.int32),
    pltpu.VMEM((RPW * 32,), jnp.int32),
    pltpu.VMEM((RPW * 64,), jnp.int32),
    pltpu.VMEM((G * 16, DP), jnp.float32),
    pltpu.VMEM((G * 32, DP), jnp.float32),
    pltpu.VMEM((G * 64, DP), jnp.float32),
    pltpu.VMEM((RPW + 16,), jnp.int32),
    pltpu.VMEM((RPW + 16,), jnp.float32),
    pltpu.VMEM((RPW + 16,), jnp.float32),
    pltpu.VMEM((RPW + 16,), jnp.float32),
    pltpu.SemaphoreType.DMA,
]

import functools as _functools


@_functools.lru_cache(maxsize=1)
def _get_sc_call():
    return pl.kernel(
        _sc_body,
        out_type=_SC_OUT,
        mesh=plsc.VectorSubcoreMesh(core_axis_name="c", subcore_axis_name="s"),
        scratch_types=_SC_SCRATCH,
        compiler_params=pltpu.CompilerParams(needs_layout_passes=False,
                                             use_tc_tiling_on_sc=False),
    )


def _tc_body(g0, g1, g2, cx,
             w00, w01, w02, w0x,
             w10, w11, w12, w1x,
             w20, w21, w22, w2x,
             wagg, bcat, out):
    cxv = cx[...]
    bc = bcat[...]
    gs = (g0, g1, g2)
    wss = ((w00, w01, w02), (w10, w11, w12), (w20, w21, w22))
    wxs = (w0x, w1x, w2x)
    feats = []
    for k in range(3):
        ns = NSS[k]
        g = gs[k][...]
        w0 = wss[k][0][...]
        c1 = w0.shape[1]
        cb = jnp.dot(cxv, wxs[k][...], preferred_element_type=jnp.float32)
        h = jnp.dot(g, w0, preferred_element_type=jnp.float32)
        b0 = bc[3 * k, :c1][None, None, :]
        h = h.reshape(RB, ns, c1) - cb[:, None, :] + b0
        h = jnp.maximum(h, 0.0).reshape(RB * ns, c1)
        for j in (1, 2):
            w = wss[k][j][...]
            c2 = w.shape[1]
            bj = bc[3 * k + j, :c2][None, :]
            h = jnp.maximum(
                jnp.dot(h, w, preferred_element_type=jnp.float32) + bj, 0.0)
        feats.append(jnp.max(h.reshape(RB, ns, h.shape[-1]), axis=1))
    f = jnp.concatenate(feats, axis=-1)
    ba = bc[9, :][None, :]
    out[...] = jnp.maximum(
        jnp.dot(f, wagg[...], preferred_element_type=jnp.float32) + ba, 0.0)


def _tc_call(g0f, g1f, g2f, nxz, ws, bcat):
    full = [pl.BlockSpec(w.shape, lambda i: (0, 0)) for w in ws]
    return pl.pallas_call(
        _tc_body,
        grid=(ROWS // RB,),
        in_specs=[
            pl.BlockSpec((RB * 16, DP), lambda i: (i, 0)),
            pl.BlockSpec((RB * 32, DP), lambda i: (i, 0)),
            pl.BlockSpec((RB * 64, DP), lambda i: (i, 0)),
            pl.BlockSpec((RB, 8), lambda i: (i, 0)),
        ] + full + [pl.BlockSpec(bcat.shape, lambda i: (0, 0))],
        out_specs=pl.BlockSpec((RB, 256), lambda i: (i, 0)),
        out_shape=jax.ShapeDtypeStruct((ROWS, 256), jnp.float32),
    )(g0f, g1f, g2f, nxz, *ws, bcat)


def kernel(xyz, points, former_fps_idx, vote_ctr,
           W0_0, b0_0, W0_1, b0_1, W0_2, b0_2,
           W1_0, b1_0, W1_1, b1_1, W1_2, b1_2,
           W2_0, b2_0, W2_1, b2_1, W2_2, b2_2,
           W_agg, b_agg):
    x = xyz[:, :, 0]
    y = xyz[:, :, 1]
    z = xyz[:, :, 2]
    featp = jnp.concatenate(
        [xyz, points, jnp.zeros((B, N, DP - CIN), jnp.float32)],
        axis=-1).reshape(B * N, DP)
    votep = jnp.pad(vote_ctr,
                    ((0, 0), (0, 0), (0, 5))).reshape(B * NPOINT * 8)
    ar = jnp.arange(NPOINT, dtype=jnp.int32)
    fps_idx = jnp.concatenate(
        [jnp.broadcast_to(ar[None], (B, NPOINT)),
         former_fps_idx.astype(jnp.int32)], axis=-1)
    vidx = fps_idx.reshape(ROWS)

    nxzp, g0f, g1f, g2f = _get_sc_call()(x, y, z, vidx, votep, featp)
    nxzt = nxzp.reshape(3, ROWS).T
    nxz = jnp.pad(nxzt, ((0, 0), (0, 5)))

    def wp0(w):
        return jnp.pad(w, ((0, 0), (0, DP - w.shape[1]))).T

    def wx(w):
        return jnp.pad(w[:, :3].T, ((0, 5), (0, 0)))

    ws = [wp0(W0_0), W0_1.T, W0_2.T, wx(W0_0),
          wp0(W1_0), W1_1.T, W1_2.T, wx(W1_0),
          wp0(W2_0), W2_1.T, W2_2.T, wx(W2_0),
          W_agg.T]
    bcat = jnp.zeros((16, 256), jnp.float32)
    for i, bb in enumerate([b0_0, b0_1, b0_2, b1_0, b1_1, b1_2,
                            b2_0, b2_1, b2_2, b_agg]):
        bcat = bcat.at[i, :bb.shape[0]].set(bb)

    feat = _tc_call(g0f, g1f, g2f, nxz, ws, bcat)
    return (nxzt.reshape(B, S, 3),
            feat.reshape(B, S, 256),
            fps_idx)


# speedup vs baseline: 26.5937x; 1.0153x over previous
import jax
import jax.numpy as jnp
from jax import lax
from jax.experimental import pallas as pl
from jax.experimental.pallas import tpu as pltpu
from jax.experimental.pallas import tpu_sc as plsc

B = 4
N = 4096
NPOINT = 512
S = 576
ROWS = B * S
DP = 48
CIN = 35
NSS = (16, 32, 64)
RSQ = (0.2 * 0.2, 0.4 * 0.4, 0.8 * 0.8)
NW = 32
RPW = ROWS // NW
G = 12
NG = RPW // G
RB = 128
CHUNKS = N // 16


def _sc_body(xh, yh, zh, vidxh, voteh, feath,
             nxzh, g0h, g1h, g2h,
             xv, yv, zv, x2v, vteb, cand0, cand1, cand2,
             land0, land1, land2, vidxv, cxr, cyr, czr, sem):
    cid = lax.axis_index("c")
    sid = lax.axis_index("s")
    wid = sid * 2 + cid
    b = wid // (S // RPW)
    base = wid * RPW
    pltpu.sync_copy(xh.at[b], xv)
    pltpu.sync_copy(yh.at[b], yv)
    pltpu.sync_copy(zh.at[b], zv)
    pltpu.sync_copy(vidxh.at[pl.ds(base, RPW)], vidxv.at[pl.ds(0, RPW)])
    pltpu.sync_copy(voteh.at[pl.ds(b * NPOINT * 8, NPOINT * 8)], vteb)
    boff = b * N
    iota = lax.iota(jnp.int32, 16)
    zero16 = jnp.zeros((16,), jnp.int32)
    boffv = jnp.broadcast_to(boff, (16,))
    cands = (cand0, cand1, cand2)

    def bf16r(v):
        u = plsc.bitcast(v, jnp.int32)
        u = (u + 0x7FFF + ((u >> 16) & 1)) & ~0xFFFF
        return plsc.bitcast(u, jnp.float32)

    def pre_body(c, _):
        sl = pl.ds(c * 16, 16)
        px, py, pz = xv[sl], yv[sl], zv[sl]
        x2v[sl] = (px * px + py * py) + pz * pz
        xv[sl] = bf16r(px)
        yv[sl] = bf16r(py)
        zv[sl] = bf16r(pz)
        return 0

    lax.fori_loop(0, CHUNKS, pre_body, 0)

    for q in range(RPW // 16 + 1):
        sl = pl.ds(q * 16, 16)
        vv = vidxv[sl]
        vv = jnp.minimum(jnp.maximum(vv, 0), NPOINT - 1) * 8
        cxr[sl] = plsc.load_gather(vteb, [vv])
        cyr[sl] = plsc.load_gather(vteb, [vv + 1])
        czr[sl] = plsc.load_gather(vteb, [vv + 2])

    def row_body(r, _):
        ridx = jnp.broadcast_to(r, (16,))
        cx = plsc.load_gather(cxr, [ridx])
        cy = plsc.load_gather(cyr, [ridx])
        cz = plsc.load_gather(czr, [ridx])
        c2 = (cx * cx + cy * cy) + cz * cz
        cxb = bf16r(cx)
        cyb = bf16r(cy)
        czb = bf16r(cz)

        candr = tuple(cands[k].at[pl.ds(r * NSS[k], NSS[k])]
                      for k in range(3))

        def chunk(c, carry):
            sl = pl.ds(c * 16, 16)
            dot = (cxb * xv[sl] + cyb * yv[sl]) + czb * zv[sl]
            d = (c2 + x2v[sl]) - 2.0 * dot
            gi = iota + (c * 16 + boff)
            new = []
            for k in range(3):
                ns = NSS[k]
                m = d <= jnp.float32(RSQ[k])
                cnt = carry[k]
                posr = cnt + plsc.cumsum(m.astype(jnp.int32)) - 1
                mst = jnp.logical_and(m, posr < ns)
                pos = jnp.minimum(posr, ns - 1)
                plsc.store_scatter(candr[k], [pos], gi, mask=mst)
                new.append(jnp.minimum(
                    cnt + plsc.all_reduce_population_count(m), ns))
            return tuple(new)

        cnts = lax.fori_loop(0, CHUNKS, chunk, (zero16, zero16, zero16),
                             unroll=4)
        for k in range(3):
            ns = NSS[k]
            cnt = cnts[k]
            first = plsc.load_gather(cands[k], [zero16 + r * ns])
            sel = jnp.where(cnt > 0, first, boffv)
            for q in range(ns // 16):
                slq = pl.ds(r * ns + q * 16, 16)
                keep = (iota + q * 16) < cnt
                cands[k][slq] = jnp.where(keep, cands[k][slq], sel)
        return 0

    lax.fori_loop(0, RPW, row_body, 0)

    ghs = (g0h, g1h, g2h)
    lands = (land0, land1, land2)

    def group(gidx, _):
        row0 = gidx * G
        cps = []
        for k in range(3):
            ns = NSS[k]
            tot = G * ns
            nch = (tot + 127) // 128
            csz = tot // nch
            for j in range(nch):
                idx_ref = cands[k].at[pl.ds(row0 * ns + j * csz, csz)]
                dst = lands[k].at[pl.ds(j * csz, csz)]
                cps.append(pltpu.async_copy(feath.at[idx_ref], dst, sem))
        for cp in cps:
            cp.wait()
        for k in range(3):
            ns = NSS[k]
            pltpu.sync_copy(lands[k],
                            ghs[k].at[pl.ds((base + row0) * ns, G * ns)])
        return 0

    lax.fori_loop(0, NG, group, 0)
    pltpu.sync_copy(cxr.at[pl.ds(0, RPW)], nxzh.at[pl.ds(base, RPW)])
    pltpu.sync_copy(cyr.at[pl.ds(0, RPW)], nxzh.at[pl.ds(ROWS + base, RPW)])
    pltpu.sync_copy(czr.at[pl.ds(0, RPW)],
                    nxzh.at[pl.ds(2 * ROWS + base, RPW)])


_SC_OUT = (
    jax.ShapeDtypeStruct((3 * ROWS,), jnp.float32),
    jax.ShapeDtypeStruct((ROWS * 16, DP), jnp.float32),
    jax.ShapeDtypeStruct((ROWS * 32, DP), jnp.float32),
    jax.ShapeDtypeStruct((ROWS * 64, DP), jnp.float32),
)

_SC_SCRATCH = [
    pltpu.VMEM((N,), jnp.float32),
    pltpu.VMEM((N,), jnp.float32),
    pltpu.VMEM((N,), jnp.float32),
    pltpu.VMEM((N,), jnp.float32),
    pltpu.VMEM((NPOINT * 8,), jnp.float32),
    pltpu.VMEM((RPW * 16,), jnp.int32),
    pltpu.VMEM((RPW * 32,), jnp.int32),
    pltpu.VMEM((RPW * 64,), jnp.int32),
    pltpu.VMEM((G * 16, DP), jnp.float32),
    pltpu.VMEM((G * 32, DP), jnp.float32),
    pltpu.VMEM((G * 64, DP), jnp.float32),
    pltpu.VMEM((RPW + 16,), jnp.int32),
    pltpu.VMEM((RPW + 16,), jnp.float32),
    pltpu.VMEM((RPW + 16,), jnp.float32),
    pltpu.VMEM((RPW + 16,), jnp.float32),
    pltpu.SemaphoreType.DMA,
]

import functools as _functools


@_functools.lru_cache(maxsize=1)
def _get_sc_call():
    return pl.kernel(
        _sc_body,
        out_type=_SC_OUT,
        mesh=plsc.VectorSubcoreMesh(core_axis_name="c", subcore_axis_name="s"),
        scratch_types=_SC_SCRATCH,
        compiler_params=pltpu.CompilerParams(needs_layout_passes=False,
                                             use_tc_tiling_on_sc=False),
    )


def _tc_body(g0, g1, g2, cx,
             w00, w01, w02, w0x,
             w10, w11, w12, w1x,
             w20, w21, w22, w2x,
             wagg, bcat, out):
    cxv = cx[...]
    bc = bcat[...]
    gs = (g0, g1, g2)
    wss = ((w00, w01, w02), (w10, w11, w12), (w20, w21, w22))
    wxs = (w0x, w1x, w2x)
    feats = []
    for k in range(3):
        ns = NSS[k]
        g = gs[k][...]
        w0 = wss[k][0][...]
        c1 = w0.shape[1]
        cb = jnp.dot(cxv, wxs[k][...], preferred_element_type=jnp.float32)
        h = jnp.dot(g, w0, preferred_element_type=jnp.float32)
        b0 = bc[3 * k, :c1][None, None, :]
        h = h.reshape(RB, ns, c1) - cb[:, None, :] + b0
        h = jnp.maximum(h, 0.0).reshape(RB * ns, c1)
        for j in (1, 2):
            w = wss[k][j][...]
            c2 = w.shape[1]
            bj = bc[3 * k + j, :c2][None, :]
            h = jnp.maximum(
                jnp.dot(h, w, preferred_element_type=jnp.float32) + bj, 0.0)
        feats.append(jnp.max(h.reshape(RB, ns, h.shape[-1]), axis=1))
    f = jnp.concatenate(feats, axis=-1)
    ba = bc[9, :][None, :]
    out[...] = jnp.maximum(
        jnp.dot(f, wagg[...], preferred_element_type=jnp.float32) + ba, 0.0)


def _tc_call(g0f, g1f, g2f, nxz, ws, bcat):
    full = [pl.BlockSpec(w.shape, lambda i: (0, 0)) for w in ws]
    return pl.pallas_call(
        _tc_body,
        grid=(ROWS // RB,),
        in_specs=[
            pl.BlockSpec((RB * 16, DP), lambda i: (i, 0)),
            pl.BlockSpec((RB * 32, DP), lambda i: (i, 0)),
            pl.BlockSpec((RB * 64, DP), lambda i: (i, 0)),
            pl.BlockSpec((RB, 8), lambda i: (i, 0)),
        ] + full + [pl.BlockSpec(bcat.shape, lambda i: (0, 0))],
        out_specs=pl.BlockSpec((RB, 256), lambda i: (i, 0)),
        out_shape=jax.ShapeDtypeStruct((ROWS, 256), jnp.float32),
    )(g0f, g1f, g2f, nxz, *ws, bcat)


def kernel(xyz, points, former_fps_idx, vote_ctr,
           W0_0, b0_0, W0_1, b0_1, W0_2, b0_2,
           W1_0, b1_0, W1_1, b1_1, W1_2, b1_2,
           W2_0, b2_0, W2_1, b2_1, W2_2, b2_2,
           W_agg, b_agg):
    x = xyz[:, :, 0]
    y = xyz[:, :, 1]
    z = xyz[:, :, 2]
    featp = jnp.concatenate(
        [xyz, points, jnp.zeros((B, N, DP - CIN), jnp.float32)],
        axis=-1).reshape(B * N, DP)
    votep = jnp.pad(vote_ctr,
                    ((0, 0), (0, 0), (0, 5))).reshape(B * NPOINT * 8)
    ar = jnp.arange(NPOINT, dtype=jnp.int32)
    fps_idx = jnp.concatenate(
        [jnp.broadcast_to(ar[None], (B, NPOINT)),
         former_fps_idx.astype(jnp.int32)], axis=-1)
    vidx = fps_idx.reshape(ROWS)

    nxzp, g0f, g1f, g2f = _get_sc_call()(x, y, z, vidx, votep, featp)
    nxzt = nxzp.reshape(3, ROWS).T
    nxz = jnp.pad(nxzt, ((0, 0), (0, 5)))

    def wp0(w):
        return jnp.pad(w, ((0, 0), (0, DP - w.shape[1]))).T

    def wx(w):
        return jnp.pad(w[:, :3].T, ((0, 5), (0, 0)))

    ws = [wp0(W0_0), W0_1.T, W0_2.T, wx(W0_0),
          wp0(W1_0), W1_1.T, W1_2.T, wx(W1_0),
          wp0(W2_0), W2_1.T, W2_2.T, wx(W2_0),
          W_agg.T]
    bcat = jnp.zeros((16, 256), jnp.float32)
    for i, bb in enumerate([b0_0, b0_1, b0_2, b1_0, b1_1, b1_2,
                            b2_0, b2_1, b2_2, b_agg]):
        bcat = bcat.at[i, :bb.shape[0]].set(bb)

    feat = _tc_call(g0f, g1f, g2f, nxz, ws, bcat)
    return (nxzt.reshape(B, S, 3),
            feat.reshape(B, S, 256),
            fps_idx)
